# baseline (speedup 1.0000x reference)
import functools

import numpy as np
import jax
from jax import lax
import jax.numpy as jnp
from jax.experimental import pallas as pl
from jax.experimental.pallas import tpu as pltpu
from jax.experimental.pallas import tpu_sc as plsc

_DROPOUT = 0.1
_UNK_ID = 0
_ROWS, _COLS = 16384, 200
_N = _ROWS * _COLS

_NC = 2
_NS = 16
_NW = _NC * _NS
_CHUNK = _N // _NW
_VBLK = 512
_BLOCKS = _CHUNK // _VBLK
_MWORDS = _N // 32
_MW_PER_W = _MWORDS // _NW


def _threefry2x32_np(k1, k2, x0, x1):
    def rotl(x, r):
        return ((x << np.uint32(r)) | (x >> np.uint32(32 - r))).astype(np.uint32)
    ks = [np.uint32(k1), np.uint32(k2),
          np.uint32(np.uint32(k1) ^ np.uint32(k2) ^ np.uint32(0x1BD11BDA))]
    x = [(x0 + ks[0]).astype(np.uint32), (x1 + ks[1]).astype(np.uint32)]
    rotations = [[13, 15, 26, 6], [17, 29, 16, 24]]
    for i in range(5):
        for r in rotations[i % 2]:
            x[0] = (x[0] + x[1]).astype(np.uint32)
            x[1] = rotl(x[1], r)
            x[1] = (x[1] ^ x[0]).astype(np.uint32)
        x[0] = (x[0] + ks[(i + 1) % 3]).astype(np.uint32)
        x[1] = (x[1] + ks[(i + 2) % 3] + np.uint32(i + 1)).astype(np.uint32)
    return x


@functools.cache
def _mask_flat() -> np.ndarray:
    i64 = np.arange(_N, dtype=np.uint64)
    hi = (i64 >> np.uint64(32)).astype(np.uint32)
    lo = (i64 & np.uint64(0xFFFFFFFF)).astype(np.uint32)
    o = _threefry2x32_np(np.uint32(0), np.uint32(42), hi, lo)
    bits = o[0] ^ o[1]
    bits = (bits >> np.uint32(9)) | np.uint32(0x3F800000)
    floats = bits.view(np.float32) - np.float32(1.0)
    u = np.maximum(np.float32(0), floats)
    return u < np.float32(_DROPOUT)


@functools.cache
def _mask_words() -> np.ndarray:
    m3 = _mask_flat().reshape(-1, 32, 16).astype(np.uint32)
    shifts = np.arange(32, dtype=np.uint32)[None, :, None]
    words = (m3 << shifts).sum(axis=1, dtype=np.uint32)
    return words.astype(np.int32).reshape(-1)


@functools.partial(
    pl.kernel,
    out_type=jax.ShapeDtypeStruct((_N,), jnp.int32),
    mesh=plsc.VectorSubcoreMesh(core_axis_name="c", subcore_axis_name="s"),
    scratch_types=[
        pltpu.VMEM((_CHUNK,), jnp.int32),
        pltpu.VMEM((_MW_PER_W,), jnp.int32),
    ],
)
def _drop_sc(x_hbm, mw_hbm, o_hbm, xbuf, mbuf):
    wid = lax.axis_index("s") * _NC + lax.axis_index("c")
    base = wid * _CHUNK
    pltpu.sync_copy(x_hbm.at[pl.ds(base, _CHUNK)], xbuf)
    pltpu.sync_copy(mw_hbm.at[pl.ds(wid * _MW_PER_W, _MW_PER_W)], mbuf)

    def blk_body(b, carry):
        mw = mbuf[pl.ds(b * 16, 16)]
        for v in range(32):
            off = b * _VBLK + v * 16
            x = xbuf[pl.ds(off, 16)]
            xbuf[pl.ds(off, 16)] = x & (((mw >> v) & 1) - 1)
        return carry

    lax.fori_loop(0, _BLOCKS, blk_body, 0)
    pltpu.sync_copy(xbuf, o_hbm.at[pl.ds(base, _CHUNK)])


def kernel(inputs):
    mw = jnp.asarray(_mask_words())
    out = _drop_sc(inputs.reshape(_N), mw)
    return out.reshape(_ROWS, _COLS)


# speedup vs baseline: 1.0018x; 1.0018x over previous
import functools

import numpy as np
import jax
from jax import lax
import jax.numpy as jnp
from jax.experimental import pallas as pl
from jax.experimental.pallas import tpu as pltpu
from jax.experimental.pallas import tpu_sc as plsc

_DROPOUT = 0.1
_UNK_ID = 0
_ROWS, _COLS = 16384, 200
_N = _ROWS * _COLS

_NC = 2
_NS = 16
_NW = _NC * _NS
_CHUNK = _N // _NW
_VBLK = 512
_BLOCKS = _CHUNK // _VBLK
_MWORDS = _N // 32
_MW_PER_W = _MWORDS // _NW


def _threefry2x32_np(k1, k2, x0, x1):
    def rotl(x, r):
        return ((x << np.uint32(r)) | (x >> np.uint32(32 - r))).astype(np.uint32)
    ks = [np.uint32(k1), np.uint32(k2),
          np.uint32(np.uint32(k1) ^ np.uint32(k2) ^ np.uint32(0x1BD11BDA))]
    x = [(x0 + ks[0]).astype(np.uint32), (x1 + ks[1]).astype(np.uint32)]
    rotations = [[13, 15, 26, 6], [17, 29, 16, 24]]
    for i in range(5):
        for r in rotations[i % 2]:
            x[0] = (x[0] + x[1]).astype(np.uint32)
            x[1] = rotl(x[1], r)
            x[1] = (x[1] ^ x[0]).astype(np.uint32)
        x[0] = (x[0] + ks[(i + 1) % 3]).astype(np.uint32)
        x[1] = (x[1] + ks[(i + 2) % 3] + np.uint32(i + 1)).astype(np.uint32)
    return x


@functools.cache
def _mask_flat() -> np.ndarray:
    i64 = np.arange(_N, dtype=np.uint64)
    hi = (i64 >> np.uint64(32)).astype(np.uint32)
    lo = (i64 & np.uint64(0xFFFFFFFF)).astype(np.uint32)
    o = _threefry2x32_np(np.uint32(0), np.uint32(42), hi, lo)
    bits = o[0] ^ o[1]
    bits = (bits >> np.uint32(9)) | np.uint32(0x3F800000)
    floats = bits.view(np.float32) - np.float32(1.0)
    u = np.maximum(np.float32(0), floats)
    return u < np.float32(_DROPOUT)


@functools.cache
def _mask_words() -> np.ndarray:
    keep = ~_mask_flat()
    m3 = keep.reshape(-1, 32, 16).astype(np.uint32)
    shifts = (np.uint32(31) - np.arange(32, dtype=np.uint32))[None, :, None]
    words = (m3 << shifts).sum(axis=1, dtype=np.uint32)
    return words.view(np.int32).reshape(-1)


@functools.partial(
    pl.kernel,
    out_type=jax.ShapeDtypeStruct((_N,), jnp.int32),
    mesh=plsc.VectorSubcoreMesh(core_axis_name="c", subcore_axis_name="s"),
    scratch_types=[
        pltpu.VMEM((_CHUNK,), jnp.int32),
        pltpu.VMEM((_MW_PER_W,), jnp.int32),
    ],
)
def _drop_sc(x_hbm, mw_hbm, o_hbm, xbuf, mbuf):
    wid = lax.axis_index("s") * _NC + lax.axis_index("c")
    base = wid * _CHUNK
    pltpu.sync_copy(x_hbm.at[pl.ds(base, _CHUNK)], xbuf)
    pltpu.sync_copy(mw_hbm.at[pl.ds(wid * _MW_PER_W, _MW_PER_W)], mbuf)

    def blk_body(b, carry):
        mw = mbuf[pl.ds(b * 16, 16)]
        for v in range(32):
            off = b * _VBLK + v * 16
            x = xbuf[pl.ds(off, 16)]
            xbuf[pl.ds(off, 16)] = x & ((mw << v) >> 31)
        return carry

    lax.fori_loop(0, _BLOCKS, blk_body, 0)
    pltpu.sync_copy(xbuf, o_hbm.at[pl.ds(base, _CHUNK)])


def kernel(inputs):
    mw = jnp.asarray(_mask_words())
    out = _drop_sc(inputs.reshape(_N), mw)
    return out.reshape(_ROWS, _COLS)
